# Initial kernel scaffold; baseline (speedup 1.0000x reference)
import jax
import jax.numpy as jnp
from jax.experimental import pallas as pl


def kernel(x, adj, W1, b1, W2, b2, W3, b3):
    raise NotImplementedError("write your pallas kernel here")


# speedup vs baseline: 3.8163x; 3.8163x over previous
import functools

import jax
import jax.numpy as jnp
from jax import lax
from jax.experimental import pallas as pl
from jax.experimental.pallas import tpu as pltpu
from jax.experimental.pallas import tpu_sc as plsc

N = 10000
D = 128
NC = 2
NS = 16
NW = NC * NS
CHUNK = 128
N_PAD = 10240
ROWS_PER_TILE = N_PAD // NS


@functools.lru_cache(maxsize=None)
def _make_scatter(e_pad):
  per_tile = e_pad // NW
  n_chunks = per_tile // CHUNK
  mesh = plsc.VectorSubcoreMesh(
      core_axis_name="c", subcore_axis_name="s", num_cores=NC, num_subcores=NS
  )

  @functools.partial(
      pl.kernel,
      out_type=jax.ShapeDtypeStruct((NC, N_PAD, D), jnp.float32),
      mesh=mesh,
      scratch_types=[
          pltpu.VMEM((CHUNK,), jnp.int32),
          pltpu.VMEM((CHUNK,), jnp.int32),
          pltpu.VMEM((CHUNK, D), jnp.float32),
          pltpu.VMEM_SHARED((N_PAD, D), jnp.float32),
          pltpu.SemaphoreType.DMA,
      ],
  )
  def scatter_kernel(g_hbm, src_hbm, dst_hbm, z_hbm, out_hbm,
                     src_v, dst_v, rows_v, acc_sh, sem):
    c = lax.axis_index("c")
    s = lax.axis_index("s")
    wid = s * NC + c

    pltpu.sync_copy(z_hbm, acc_sh.at[pl.ds(s * ROWS_PER_TILE, ROWS_PER_TILE)])
    plsc.subcore_barrier()

    base0 = wid * per_tile

    def step(j, carry):
      base = base0 + j * CHUNK
      pltpu.sync_copy(src_hbm.at[pl.ds(base, CHUNK)], src_v)
      pltpu.sync_copy(dst_hbm.at[pl.ds(base, CHUNK)], dst_v)
      pltpu.async_copy(g_hbm.at[src_v], rows_v, sem).wait()
      pltpu.sync_copy(rows_v, acc_sh.at[dst_v], add=True)
      return carry

    lax.fori_loop(0, n_chunks, step, 0)
    plsc.subcore_barrier()

    pltpu.sync_copy(
        acc_sh.at[pl.ds(s * ROWS_PER_TILE, ROWS_PER_TILE)],
        out_hbm.at[c, pl.ds(s * ROWS_PER_TILE, ROWS_PER_TILE)],
    )

  return scatter_kernel


_BM = 1024


def _mm0_body(x_ref, w_ref, o_ref):
  o_ref[...] = jnp.dot(x_ref[...], w_ref[...], preferred_element_type=jnp.float32)


def _mm0(x, w):
  grid = x.shape[0] // _BM
  return pl.pallas_call(
      _mm0_body,
      grid=(grid,),
      in_specs=[
          pl.BlockSpec((_BM, D), lambda i: (i, 0)),
          pl.BlockSpec((D, D), lambda i: (0, 0)),
      ],
      out_specs=pl.BlockSpec((_BM, D), lambda i: (i, 0)),
      out_shape=jax.ShapeDtypeStruct((x.shape[0], D), jnp.float32),
  )(x, w)


def _mm_mid_body(p0_ref, p1_ref, b_ref, w_ref, o_ref):
  h = jnp.maximum(p0_ref[...] + p1_ref[...] + b_ref[...], 0.0)
  o_ref[...] = jnp.dot(h, w_ref[...], preferred_element_type=jnp.float32)


def _mm_mid(p0, p1, b, w):
  grid = p0.shape[0] // _BM
  return pl.pallas_call(
      _mm_mid_body,
      grid=(grid,),
      in_specs=[
          pl.BlockSpec((_BM, D), lambda i: (i, 0)),
          pl.BlockSpec((_BM, D), lambda i: (i, 0)),
          pl.BlockSpec((1, D), lambda i: (0, 0)),
          pl.BlockSpec((D, D), lambda i: (0, 0)),
      ],
      out_specs=pl.BlockSpec((_BM, D), lambda i: (i, 0)),
      out_shape=jax.ShapeDtypeStruct((p0.shape[0], D), jnp.float32),
  )(p0, p1, b, w)


_BF = 1000


def _fin_body(p0_ref, p1_ref, b_ref, o_ref):
  h = p0_ref[...] + p1_ref[...] + b_ref[...]
  nrm = jnp.sqrt(jnp.sum(h * h, axis=1, keepdims=True))
  o_ref[...] = h / jnp.maximum(nrm, 1e-12)


def _fin(p0, p1, b):
  return pl.pallas_call(
      _fin_body,
      grid=(N // _BF,),
      in_specs=[
          pl.BlockSpec((_BF, D), lambda i: (i, 0)),
          pl.BlockSpec((_BF, D), lambda i: (i, 0)),
          pl.BlockSpec((1, D), lambda i: (0, 0)),
      ],
      out_specs=pl.BlockSpec((_BF, D), lambda i: (i, 0)),
      out_shape=jax.ShapeDtypeStruct((N, D), jnp.float32),
  )(p0, p1, b)


def kernel(x, adj, W1, b1, W2, b2, W3, b3):
  e = adj.shape[1]
  e_pad = ((e + NW * CHUNK - 1) // (NW * CHUNK)) * NW * CHUNK
  pad = e_pad - e

  src = jnp.concatenate([adj[0], jnp.zeros((pad,), jnp.int32)])
  dst = jnp.concatenate(
      [adj[1], N + (jnp.arange(pad, dtype=jnp.int32) % (N_PAD - N))]
  )
  xp = jnp.concatenate([x, jnp.zeros((N_PAD - N, D), jnp.float32)])
  zeros = jnp.zeros((ROWS_PER_TILE, D), jnp.float32)

  scatter = _make_scatter(e_pad)

  g = _mm0(xp, W1)
  p = scatter(g, src, dst, zeros)
  g = _mm_mid(p[0], p[1], b1.reshape(1, D), W2)
  p = scatter(g, src, dst, zeros)
  g = _mm_mid(p[0], p[1], b2.reshape(1, D), W3)
  p = scatter(g, src, dst, zeros)
  return _fin(p[0], p[1], b3.reshape(1, D))


# speedup vs baseline: 4.4618x; 1.1691x over previous
import functools

import jax
import jax.numpy as jnp
from jax import lax
from jax.experimental import pallas as pl
from jax.experimental.pallas import tpu as pltpu
from jax.experimental.pallas import tpu_sc as plsc

N = 10000
D = 128
NC = 2
NS = 16
NW = NC * NS
CHUNK = 128
N_PAD = 10240
ROWS_PER_TILE = N_PAD // NS


_K = 1


@functools.lru_cache(maxsize=None)
def _make_scatter(e_pad):
  per_tile = e_pad // NW
  n_chunks = per_tile // CHUNK
  assert n_chunks % _K == 0
  laps = n_chunks // _K
  mesh = plsc.VectorSubcoreMesh(
      core_axis_name="c", subcore_axis_name="s", num_cores=NC, num_subcores=NS
  )

  @functools.partial(
      pl.kernel,
      out_type=jax.ShapeDtypeStruct((NC, N_PAD, D), jnp.float32),
      mesh=mesh,
      scratch_types=[
          pltpu.VMEM((per_tile,), jnp.int32),
          pltpu.VMEM((per_tile,), jnp.int32),
          pltpu.VMEM((_K, CHUNK, D), jnp.float32),
          pltpu.VMEM_SHARED((N_PAD, D), jnp.float32),
          [pltpu.SemaphoreType.DMA] * _K,
      ],
  )
  def scatter_kernel(g_hbm, src_hbm, dst_hbm, z_hbm, out_hbm,
                     src_v, dst_v, rows_v, acc_sh, gsems):
    c = lax.axis_index("c")
    s = lax.axis_index("s")
    wid = s * NC + c
    base0 = wid * per_tile

    pltpu.sync_copy(src_hbm.at[pl.ds(base0, per_tile)], src_v)
    pltpu.sync_copy(dst_hbm.at[pl.ds(base0, per_tile)], dst_v)

    def sidx(ref, j):
      return ref.at[pl.ds(j * CHUNK, CHUNK)]

    gd = [
        pltpu.async_copy(g_hbm.at[sidx(src_v, b)], rows_v.at[b], gsems[b])
        for b in range(_K)
    ]
    pltpu.sync_copy(z_hbm, acc_sh.at[pl.ds(s * ROWS_PER_TILE, ROWS_PER_TILE)])
    plsc.subcore_barrier()

    def lap(i, carry):
      for b in range(_K):
        j = i * _K + b
        pltpu.make_async_copy(g_hbm.at[sidx(src_v, j)], rows_v.at[b],
                              gsems[b]).wait()
        pltpu.sync_copy(rows_v.at[b], acc_sh.at[sidx(dst_v, j)], add=True)

        @pl.when(i < laps - 1)
        def _(b=b):
          pltpu.async_copy(g_hbm.at[sidx(src_v, (i + 1) * _K + b)],
                           rows_v.at[b], gsems[b])

      return carry

    lax.fori_loop(0, laps, lap, 0)
    plsc.subcore_barrier()

    pltpu.sync_copy(
        acc_sh.at[pl.ds(s * ROWS_PER_TILE, ROWS_PER_TILE)],
        out_hbm.at[c, pl.ds(s * ROWS_PER_TILE, ROWS_PER_TILE)],
    )

  return scatter_kernel


_BM = 1024


def _mm0_body(x_ref, w_ref, o_ref):
  o_ref[...] = jnp.dot(x_ref[...], w_ref[...], preferred_element_type=jnp.float32)


def _mm0(x, w):
  grid = x.shape[0] // _BM
  return pl.pallas_call(
      _mm0_body,
      grid=(grid,),
      in_specs=[
          pl.BlockSpec((_BM, D), lambda i: (i, 0)),
          pl.BlockSpec((D, D), lambda i: (0, 0)),
      ],
      out_specs=pl.BlockSpec((_BM, D), lambda i: (i, 0)),
      out_shape=jax.ShapeDtypeStruct((x.shape[0], D), jnp.float32),
  )(x, w)


def _mm_mid_body(p0_ref, p1_ref, b_ref, w_ref, o_ref):
  h = jnp.maximum(p0_ref[...] + p1_ref[...] + b_ref[...], 0.0)
  o_ref[...] = jnp.dot(h, w_ref[...], preferred_element_type=jnp.float32)


def _mm_mid(p0, p1, b, w):
  grid = p0.shape[0] // _BM
  return pl.pallas_call(
      _mm_mid_body,
      grid=(grid,),
      in_specs=[
          pl.BlockSpec((_BM, D), lambda i: (i, 0)),
          pl.BlockSpec((_BM, D), lambda i: (i, 0)),
          pl.BlockSpec((1, D), lambda i: (0, 0)),
          pl.BlockSpec((D, D), lambda i: (0, 0)),
      ],
      out_specs=pl.BlockSpec((_BM, D), lambda i: (i, 0)),
      out_shape=jax.ShapeDtypeStruct((p0.shape[0], D), jnp.float32),
  )(p0, p1, b, w)


_BF = 1000


def _fin_body(p0_ref, p1_ref, b_ref, o_ref):
  h = p0_ref[...] + p1_ref[...] + b_ref[...]
  nrm = jnp.sqrt(jnp.sum(h * h, axis=1, keepdims=True))
  o_ref[...] = h / jnp.maximum(nrm, 1e-12)


def _fin(p0, p1, b):
  return pl.pallas_call(
      _fin_body,
      grid=(N // _BF,),
      in_specs=[
          pl.BlockSpec((_BF, D), lambda i: (i, 0)),
          pl.BlockSpec((_BF, D), lambda i: (i, 0)),
          pl.BlockSpec((1, D), lambda i: (0, 0)),
      ],
      out_specs=pl.BlockSpec((_BF, D), lambda i: (i, 0)),
      out_shape=jax.ShapeDtypeStruct((N, D), jnp.float32),
  )(p0, p1, b)


def kernel(x, adj, W1, b1, W2, b2, W3, b3):
  e = adj.shape[1]
  gran = NW * CHUNK * _K
  e_pad = ((e + gran - 1) // gran) * gran
  pad = e_pad - e

  src = jnp.concatenate([adj[0], jnp.zeros((pad,), jnp.int32)])
  dst = jnp.concatenate(
      [adj[1], N + (jnp.arange(pad, dtype=jnp.int32) % (N_PAD - N))]
  )
  xp = jnp.concatenate([x, jnp.zeros((N_PAD - N, D), jnp.float32)])
  zeros = jnp.zeros((ROWS_PER_TILE, D), jnp.float32)

  scatter = _make_scatter(e_pad)

  g = _mm0(xp, W1)
  p = scatter(g, src, dst, zeros)
  g = _mm_mid(p[0], p[1], b1.reshape(1, D), W2)
  p = scatter(g, src, dst, zeros)
  g = _mm_mid(p[0], p[1], b2.reshape(1, D), W3)
  p = scatter(g, src, dst, zeros)
  return _fin(p[0], p[1], b3.reshape(1, D))
